# baseline (speedup 1.0000x reference)
import jax
import jax.numpy as jnp
from jax import lax
from jax.experimental import pallas as pl
from jax.experimental.pallas import tpu as pltpu

_ROWS = 2048


def _tc_body(pos_ref, x_ref, table_ref, o_ref):
    pos = pos_ref[0, 0]
    iota_k = lax.broadcasted_iota(jnp.int32, (_ROWS, table_ref.shape[0]), 1)
    onehot = jnp.where(iota_k == pos[:, None],
                       jnp.float32(1), jnp.float32(0)).astype(jnp.bfloat16)
    acc = jnp.dot(onehot, table_ref[...],
                  preferred_element_type=jnp.float32)
    o_ref[...] = x_ref[...] + acc


def tc_kernel(inputs, inputs_positions, pos_embedding):
    b, s, d = inputs.shape
    n = b * s
    x = inputs.reshape(n, d)
    v = pos_embedding.shape[1]
    table = pos_embedding.reshape(v, d).astype(jnp.bfloat16)
    pos = inputs_positions.astype(jnp.int32).reshape(n // _ROWS, 1, _ROWS)
    grid = n // _ROWS
    out = pl.pallas_call(
        _tc_body,
        grid=(grid,),
        in_specs=[
            pl.BlockSpec((1, 1, _ROWS), lambda i: (i, 0, 0)),
            pl.BlockSpec((_ROWS, d), lambda i: (i, 0)),
            pl.BlockSpec((v, d), lambda i: (0, 0)),
        ],
        out_specs=pl.BlockSpec((_ROWS, d), lambda i: (i, 0)),
        out_shape=jax.ShapeDtypeStruct((n, d), jnp.float32),
        compiler_params=pltpu.CompilerParams(
            dimension_semantics=("arbitrary",)),
    )(pos, x, table)
    return out.reshape(b, s, d)


kernel = tc_kernel
